# baseline (speedup 1.0000x reference)
import jax
import jax.numpy as jnp
from jax import lax
from jax.experimental import pallas as pl
from jax.experimental.pallas import tpu as pltpu
from jax.experimental.pallas import tpu_sc as plsc

NUM_CORES = 2
NUM_SUBCORES = 16
NW = NUM_CORES * NUM_SUBCORES

AW = 32768


def _make_gather_kernel(S, B, D, Vpos):
    assert B % NW == 0
    bw = B // NW
    mesh = plsc.VectorSubcoreMesh(core_axis_name="c", subcore_axis_name="s")

    def body(tok_hbm, table_hbm, out_hbm, idx_v, rows_v, gsem, wsem0, wsem1):
        wid = lax.axis_index("s") * NUM_CORES + lax.axis_index("c")
        b0 = wid * bw
        wsems = [wsem0, wsem1]
        pltpu.sync_copy(tok_hbm.at[:, pl.ds(b0, bw)], idx_v)

        def outer(t, carry):
            for p in range(2):
                s = t * 2 + p

                @pl.when(t > 0)
                def _():
                    pltpu.make_async_copy(
                        rows_v.at[p],
                        out_hbm.at[0, pl.ds(b0, bw), :],
                        wsems[p],
                    ).wait()

                pltpu.async_copy(
                    table_hbm.at[idx_v.at[s]], rows_v.at[p], gsem
                ).wait()
                pltpu.async_copy(
                    rows_v.at[p], out_hbm.at[s, pl.ds(b0, bw), :], wsems[p]
                )
            return carry

        lax.fori_loop(0, S // 2, outer, 0)
        for p in range(2):
            pltpu.make_async_copy(
                rows_v.at[p], out_hbm.at[0, pl.ds(b0, bw), :], wsems[p]
            ).wait()

    return pl.kernel(
        body,
        out_type=jax.ShapeDtypeStruct((S, B, D), jnp.float32),
        mesh=mesh,
        scratch_types=[
            pltpu.VMEM((S, bw), jnp.int32),
            pltpu.VMEM((2, bw, D), jnp.float32),
            pltpu.SemaphoreType.DMA,
            pltpu.SemaphoreType.DMA,
            pltpu.SemaphoreType.DMA,
        ],
        compiler_params=pltpu.CompilerParams(use_tc_tiling_on_sc=False),
    )


def _make_table_transpose(V, D):
    rb = AW // (128 // D)
    nblk = pl.cdiv(V, AW)

    def body(x_ref, o_ref):
        t = jnp.swapaxes(x_ref[...], 0, 1)
        t4 = t.reshape(128 // D, rb, D)
        o_ref[...] = jnp.concatenate([t4[i] for i in range(128 // D)], axis=1)

    return pl.pallas_call(
        body,
        grid=(nblk,),
        in_specs=[pl.BlockSpec((D, AW), lambda j: (0, j))],
        out_specs=pl.BlockSpec((rb, 128), lambda j: (j, 0)),
        out_shape=jax.ShapeDtypeStruct((nblk * rb, 128), jnp.float32),
    )


def _make_out_transpose(S, B, D):
    rbc = B * D // 128
    r = 128 // D

    def body(x_ref, o_ref):
        z = jnp.swapaxes(x_ref[...], 0, 1)
        z3 = z.reshape(r, D, rbc)
        o_ref[...] = jnp.concatenate([z3[i] for i in range(r)], axis=1)[None]

    return pl.pallas_call(
        body,
        grid=(S,),
        in_specs=[pl.BlockSpec((rbc, 128), lambda s: (s, 0))],
        out_specs=pl.BlockSpec((1, D, B), lambda s: (s, 0, 0)),
        out_shape=jax.ShapeDtypeStruct((S, D, B), jnp.float32),
    )


def kernel(token_ids, E):
    Bt, S = token_ids.shape
    V, D = E.shape
    r = 128 // D
    rb = AW // r
    rbc = Bt // r

    tok_t = jnp.swapaxes(token_ids.astype(jnp.int32), 0, 1)
    tok_p = jnp.transpose(tok_t.reshape(S, r, rbc), (0, 2, 1)).reshape(S, Bt)
    u = tok_p % AW
    idx = (tok_p - u) + (r * (u % rb) + u // rb).astype(jnp.int32)

    et = jnp.swapaxes(E, 0, 1)
    tbl = _make_table_transpose(V, D)(et)
    table = tbl.reshape(tbl.shape[0] * r, D)
    s1 = (S // 2 + 1) & ~1
    parts = []
    for lo, hi in ((0, s1), (s1, S)):
        ss = hi - lo
        midp = _make_gather_kernel(ss, Bt, D, table.shape[0])(idx[lo:hi], table)
        x2p = midp.reshape(ss * Bt * D // 128, 128)
        parts.append(_make_out_transpose(ss, Bt, D)(x2p))
    out3 = jnp.concatenate(parts, axis=0)
    return jnp.transpose(out3, (2, 0, 1))


# speedup vs baseline: 1.1080x; 1.1080x over previous
import jax
import jax.numpy as jnp
from jax import lax
from jax.experimental import pallas as pl
from jax.experimental.pallas import tpu as pltpu
from jax.experimental.pallas import tpu_sc as plsc

NUM_CORES = 2
NUM_SUBCORES = 16
NW = NUM_CORES * NUM_SUBCORES

AW = 32768


def _make_gather_kernel(S, B, D, Vpos):
    assert B % NW == 0
    bw = B // NW
    mesh = plsc.VectorSubcoreMesh(core_axis_name="c", subcore_axis_name="s")

    def body(tok_hbm, table_hbm, out_hbm, idx_v, rows_v, gsem, wsem0, wsem1):
        wid = lax.axis_index("s") * NUM_CORES + lax.axis_index("c")
        b0 = wid * bw
        wsems = [wsem0, wsem1]
        pltpu.sync_copy(tok_hbm.at[:, pl.ds(b0, bw)], idx_v)

        def outer(t, carry):
            for p in range(2):
                s = t * 2 + p

                @pl.when(t > 0)
                def _():
                    pltpu.make_async_copy(
                        rows_v.at[p],
                        out_hbm.at[0, pl.ds(b0, bw), :],
                        wsems[p],
                    ).wait()

                pltpu.async_copy(
                    table_hbm.at[idx_v.at[s]], rows_v.at[p], gsem
                ).wait()
                pltpu.async_copy(
                    rows_v.at[p], out_hbm.at[s, pl.ds(b0, bw), :], wsems[p]
                )
            return carry

        lax.fori_loop(0, S // 2, outer, 0)
        for p in range(2):
            pltpu.make_async_copy(
                rows_v.at[p], out_hbm.at[0, pl.ds(b0, bw), :], wsems[p]
            ).wait()

    return pl.kernel(
        body,
        out_type=jax.ShapeDtypeStruct((S, B, D), jnp.float32),
        mesh=mesh,
        scratch_types=[
            pltpu.VMEM((S, bw), jnp.int32),
            pltpu.VMEM((2, bw, D), jnp.float32),
            pltpu.SemaphoreType.DMA,
            pltpu.SemaphoreType.DMA,
            pltpu.SemaphoreType.DMA,
        ],
        compiler_params=pltpu.CompilerParams(use_tc_tiling_on_sc=False),
    )


def _make_table_transpose(V, D):
    rb = AW // (128 // D)
    nblk = pl.cdiv(V, AW)

    def body(x_ref, o_ref):
        t = jnp.swapaxes(x_ref[...], 0, 1)
        t4 = t.reshape(128 // D, rb, D)
        o_ref[...] = jnp.concatenate([t4[i] for i in range(128 // D)], axis=1)

    return pl.pallas_call(
        body,
        grid=(nblk,),
        in_specs=[pl.BlockSpec((D, AW), lambda j: (0, j))],
        out_specs=pl.BlockSpec((rb, 128), lambda j: (j, 0)),
        out_shape=jax.ShapeDtypeStruct((nblk * rb, 128), jnp.float32),
    )


def _make_out_transpose(S, B, D):
    rbc = B * D // 128
    r = 128 // D

    def body(x_ref, o_ref):
        z = jnp.swapaxes(x_ref[...], 0, 1)
        z3 = z.reshape(r, D, rbc)
        o_ref[...] = jnp.concatenate([z3[i] for i in range(r)], axis=1)[None]

    return pl.pallas_call(
        body,
        grid=(S,),
        in_specs=[pl.BlockSpec((rbc, 128), lambda s: (s, 0))],
        out_specs=pl.BlockSpec((1, D, B), lambda s: (s, 0, 0)),
        out_shape=jax.ShapeDtypeStruct((S, D, B), jnp.float32),
    )


def kernel(token_ids, E):
    Bt, S = token_ids.shape
    V, D = E.shape
    r = 128 // D
    rb = AW // r
    rbc = Bt // r

    tok_t = jnp.swapaxes(token_ids.astype(jnp.int32), 0, 1)
    tok_p = jnp.transpose(tok_t.reshape(S, r, rbc), (0, 2, 1)).reshape(S, Bt)
    u = tok_p % AW
    idx = (tok_p - u) + (r * (u % rb) + u // rb).astype(jnp.int32)

    et = jnp.swapaxes(E, 0, 1)
    tbl = _make_table_transpose(V, D)(et)
    table = tbl.reshape(tbl.shape[0] * r, D)
    mid = _make_gather_kernel(S, Bt, D, table.shape[0])(idx, table)
    x2 = mid.reshape(S * Bt * D // 128, 128)
    out3 = _make_out_transpose(S, Bt, D)(x2)
    return jnp.transpose(out3, (2, 0, 1))
